# baseline (speedup 1.0000x reference)
import functools

import jax
import jax.numpy as jnp
from jax import lax
from jax.experimental import pallas as pl
from jax.experimental.pallas import tpu as pltpu
from jax.experimental.pallas import tpu_sc as plsc

_N = 10000
_E = 320000
_D = 128
_NC = 2
_NS = 16
_NW = _NC * _NS
_CH = 128
_NCHUNK = 80
_WIN = 8
_NWIN = _NCHUNK // _WIN
_WPW = _NCHUNK * _CH
_EPAD = _NW * _WPW
_NDUMMY = _N
_NPAD = 10240
_RPT = _NPAD // _NS

_mesh = plsc.VectorSubcoreMesh(core_axis_name="c", subcore_axis_name="s")


def _sc_hist_body(edges, zeros1, deg_out, src_idx, dst_idx, ones_v,
                  hist_src, hist_dst):
    c = lax.axis_index("c")
    s = lax.axis_index("s")
    wid = s * _NC + c
    for i in range(_CH // 16):
        ones_v[pl.ds(i * 16, 16)] = jnp.ones((16,), jnp.float32)
    @pl.when(s == 0)
    def _():
        pltpu.sync_copy(zeros1, hist_src)
        pltpu.sync_copy(zeros1, hist_dst)
    pltpu.sync_copy(edges.at[0, wid], src_idx)
    pltpu.sync_copy(edges.at[1, wid], dst_idx)
    plsc.subcore_barrier()

    @pl.loop(0, _NCHUNK)
    def _(j):
        pltpu.sync_copy(ones_v, hist_src.at[src_idx.at[j]], add=True)
        pltpu.sync_copy(ones_v, hist_dst.at[dst_idx.at[j]], add=True)

    plsc.subcore_barrier()
    @pl.when(s == 0)
    def _():
        pltpu.sync_copy(hist_src, deg_out.at[c, 0])
        pltpu.sync_copy(hist_dst, deg_out.at[c, 1])


_sc_hist = pl.kernel(
    _sc_hist_body,
    out_type=jax.ShapeDtypeStruct((_NC, 2, _NPAD), jnp.float32),
    mesh=_mesh,
    scratch_types=[
        pltpu.VMEM((_NCHUNK, _CH), jnp.int32),
        pltpu.VMEM((_NCHUNK, _CH), jnp.int32),
        pltpu.VMEM((_CH,), jnp.float32),
        pltpu.VMEM_SHARED((_NPAD,), jnp.float32),
        pltpu.VMEM_SHARED((_NPAD,), jnp.float32),
    ],
)


_HALF = _NCHUNK // 2


def _sc_agg_body(h, edges, zrow, out, src_idx, dst_idx, rows, agg_sh, gsem):
    c = lax.axis_index("c")
    s = lax.axis_index("s")
    wid = s * _NC + c
    pltpu.sync_copy(zrow, agg_sh.at[pl.ds(s * _RPT, _RPT)])
    pltpu.sync_copy(edges.at[0, wid], src_idx)
    pltpu.sync_copy(edges.at[1, wid], dst_idx)
    plsc.subcore_barrier()

    @pl.loop(0, _NCHUNK)
    def _(j):
        pltpu.async_copy(h.at[src_idx.at[j]], rows, gsem).wait()
        pltpu.sync_copy(rows, agg_sh.at[dst_idx.at[j]], add=True)

    plsc.subcore_barrier()
    pltpu.sync_copy(agg_sh.at[pl.ds(s * _RPT, _RPT)],
                    out.at[c, pl.ds(s * _RPT, _RPT)])


_sc_agg = pl.kernel(
    _sc_agg_body,
    out_type=jax.ShapeDtypeStruct((_NC, _NPAD, _D), jnp.float32),
    mesh=_mesh,
    scratch_types=[
        pltpu.VMEM((_NCHUNK, _CH), jnp.int32),
        pltpu.VMEM((_NCHUNK, _CH), jnp.int32),
        pltpu.VMEM((_CH, _D), jnp.float32),
        pltpu.VMEM_SHARED((_NPAD, _D), jnp.float32),
        pltpu.SemaphoreType.DMA,
    ],
)


def _tc_proj_body(feat_ref, degp_ref, wmu_ref, wlsd_ref, epsw_ref, h_ref):
    deg = degp_ref[0] + degp_ref[1]
    norm_l = lax.rsqrt(jnp.maximum(deg, 1.0))
    w = wmu_ref[...] + jnp.exp(wlsd_ref[...]) * epsw_ref[...]
    h_ref[...] = jnp.dot(feat_ref[...] * norm_l, w,
                         preferred_element_type=jnp.float32)


_BR1 = 1280

_tc_proj = pl.pallas_call(
    _tc_proj_body,
    grid=(_NPAD // _BR1,),
    in_specs=[
        pl.BlockSpec((_BR1, _D), lambda i: (i, 0)),
        pl.BlockSpec((2, _BR1, 1), lambda i: (0, i, 0)),
        pl.BlockSpec((_D, _D), lambda i: (0, 0)),
        pl.BlockSpec((_D, _D), lambda i: (0, 0)),
        pl.BlockSpec((_D, _D), lambda i: (0, 0)),
    ],
    out_specs=pl.BlockSpec((_BR1, _D), lambda i: (i, 0)),
    out_shape=jax.ShapeDtypeStruct((_NPAD, _D), jnp.float32),
)


def _kl_sum(mu_q, logsd_q, mu_p, logsd_p):
    var_q = jnp.exp(2.0 * logsd_q)
    var_p = jnp.exp(2.0 * logsd_p)
    t = (logsd_p - logsd_q) + (var_q + (mu_q - mu_p) ** 2) / (2.0 * var_p) - 0.5
    return jnp.sum(t)


def _tc_post_body(partials_ref, degp_ref, bmu_ref, blsd_ref, epsb_ref,
                  wmu_ref, wlsd_ref, wpmu_ref, wplsd_ref, bpmu_ref,
                  bplsd_ref, rst_ref, kl_ref):
    i = pl.program_id(0)
    agg = partials_ref[0] + partials_ref[1]
    deg = degp_ref[0] + degp_ref[1]
    norm_r = lax.rsqrt(jnp.maximum(deg, 1.0))
    bias = bmu_ref[...] + jnp.exp(blsd_ref[...]) * epsb_ref[...]
    rst_ref[...] = agg * norm_r + bias

    @pl.when(i == 0)
    def _():
        kl = _kl_sum(wmu_ref[...], wlsd_ref[...], wpmu_ref[...], wplsd_ref[...])
        kl += _kl_sum(bmu_ref[...], blsd_ref[...], bpmu_ref[...], bplsd_ref[...])
        kl_ref[...] = jnp.broadcast_to(kl, (1, 1))


_BR2 = 1000

_tc_post = pl.pallas_call(
    _tc_post_body,
    grid=(_N // _BR2,),
    in_specs=[
        pl.BlockSpec((2, _BR2, _D), lambda i: (0, i, 0)),
        pl.BlockSpec((2, _BR2, 1), lambda i: (0, i, 0)),
        pl.BlockSpec((1, _D), lambda i: (0, 0)),
        pl.BlockSpec((1, _D), lambda i: (0, 0)),
        pl.BlockSpec((1, _D), lambda i: (0, 0)),
        pl.BlockSpec((_D, _D), lambda i: (0, 0)),
        pl.BlockSpec((_D, _D), lambda i: (0, 0)),
        pl.BlockSpec((_D, _D), lambda i: (0, 0)),
        pl.BlockSpec((_D, _D), lambda i: (0, 0)),
        pl.BlockSpec((1, _D), lambda i: (0, 0)),
        pl.BlockSpec((1, _D), lambda i: (0, 0)),
    ],
    out_specs=[
        pl.BlockSpec((_BR2, _D), lambda i: (i, 0)),
        pl.BlockSpec((1, 1), lambda i: (0, 0)),
    ],
    out_shape=[
        jax.ShapeDtypeStruct((_N, _D), jnp.float32),
        jax.ShapeDtypeStruct((1, 1), jnp.float32),
    ],
)


def kernel(feat, edge_index, weight_mu, weight_logsd, bias_mu, bias_logsd,
           weight_prior_mu, weight_prior_logsd, bias_prior_mu,
           bias_prior_logsd, eps_w, eps_b):
    pad = jnp.full((2, _EPAD - _E), _NDUMMY, jnp.int32)
    e = jnp.concatenate([edge_index, pad], axis=1)
    edges = e.reshape(2, _NW, _NCHUNK, _CH)
    feat_pad = jnp.concatenate(
        [feat, jnp.zeros((_NPAD - _N, _D), feat.dtype)], axis=0)
    zeros1 = jnp.zeros((_NPAD,), jnp.float32)
    zrow = jnp.zeros((_RPT, _D), jnp.float32)

    degp = _sc_hist(edges, zeros1)
    deg_src = degp[:, 0, :].reshape(_NC, _NPAD, 1)
    deg_dst = degp[:, 1, :].reshape(_NC, _NPAD, 1)

    h = _tc_proj(feat_pad, deg_src, weight_mu, weight_logsd, eps_w)
    partials = _sc_agg(h, edges, zrow)

    rst, kl = _tc_post(partials, deg_dst, bias_mu, bias_logsd, eps_b,
                       weight_mu, weight_logsd, weight_prior_mu,
                       weight_prior_logsd, bias_prior_mu, bias_prior_logsd)
    return rst, kl[0, 0]


# speedup vs baseline: 2.4576x; 2.4576x over previous
import functools

import jax
import jax.numpy as jnp
from jax import lax
from jax.experimental import pallas as pl
from jax.experimental.pallas import tpu as pltpu
from jax.experimental.pallas import tpu_sc as plsc

_N = 10000
_E = 320000
_D = 128
_NC = 2
_NS = 16
_NW = _NC * _NS
_CH = 128
_NCHUNK = 80
_WIN = 8
_NWIN = _NCHUNK // _WIN
_WPW = _NCHUNK * _CH
_EPAD = _NW * _WPW
_NDUMMY = _N
_NPAD = 10240
_RPT = _NPAD // _NS

_mesh = plsc.VectorSubcoreMesh(core_axis_name="c", subcore_axis_name="s")


def _sc_hist_body(edges, zeros1, deg_out, src_idx, dst_idx, ones_v,
                  hist_src, hist_dst):
    c = lax.axis_index("c")
    s = lax.axis_index("s")
    wid = s * _NC + c
    for i in range(_CH // 16):
        ones_v[pl.ds(i * 16, 16)] = jnp.ones((16,), jnp.float32)
    @pl.when(s == 0)
    def _():
        pltpu.sync_copy(zeros1, hist_src)
        pltpu.sync_copy(zeros1, hist_dst)
    pltpu.sync_copy(edges.at[0, wid], src_idx)
    pltpu.sync_copy(edges.at[1, wid], dst_idx)
    plsc.subcore_barrier()

    @pl.loop(0, _NCHUNK)
    def _(j):
        pltpu.sync_copy(ones_v, hist_src.at[src_idx.at[j]], add=True)
        pltpu.sync_copy(ones_v, hist_dst.at[dst_idx.at[j]], add=True)

    plsc.subcore_barrier()
    @pl.when(s == 0)
    def _():
        pltpu.sync_copy(hist_src, deg_out.at[c, 0])
        pltpu.sync_copy(hist_dst, deg_out.at[c, 1])


_sc_hist = pl.kernel(
    _sc_hist_body,
    out_type=jax.ShapeDtypeStruct((_NC, 2, _NPAD), jnp.float32),
    mesh=_mesh,
    scratch_types=[
        pltpu.VMEM((_NCHUNK, _CH), jnp.int32),
        pltpu.VMEM((_NCHUNK, _CH), jnp.int32),
        pltpu.VMEM((_CH,), jnp.float32),
        pltpu.VMEM_SHARED((_NPAD,), jnp.float32),
        pltpu.VMEM_SHARED((_NPAD,), jnp.float32),
    ],
)


_HALF = _NCHUNK // 2


def _sc_agg_body(h, edges, zrow, out, src_idx, dst_idx, rows, agg_sh, gsem):
    c = lax.axis_index("c")
    s = lax.axis_index("s")
    wid = s * _NC + c
    pltpu.sync_copy(zrow, agg_sh.at[pl.ds(s * _RPT, _RPT)])
    pltpu.sync_copy(edges.at[0, wid], src_idx)
    pltpu.sync_copy(edges.at[1, wid], dst_idx)
    plsc.subcore_barrier()

    @pl.loop(0, _NCHUNK)
    def _(j):
        pltpu.async_copy(h.at[src_idx.at[j]], rows, gsem).wait()
        pltpu.sync_copy(rows, agg_sh.at[dst_idx.at[j]], add=True)

    plsc.subcore_barrier()
    pltpu.sync_copy(agg_sh.at[pl.ds(s * _RPT, _RPT)],
                    out.at[c, pl.ds(s * _RPT, _RPT)])


_sc_agg = pl.kernel(
    _sc_agg_body,
    out_type=jax.ShapeDtypeStruct((_NC, _NPAD, _D), jnp.float32),
    mesh=_mesh,
    scratch_types=[
        pltpu.VMEM((_NCHUNK, _CH), jnp.int32),
        pltpu.VMEM((_NCHUNK, _CH), jnp.int32),
        pltpu.VMEM((_CH, _D), jnp.float32),
        pltpu.VMEM_SHARED((_NPAD, _D), jnp.float32),
        pltpu.SemaphoreType.DMA,
    ],
)


def _tc_proj_body(feat_ref, degp_ref, wmu_ref, wlsd_ref, epsw_ref, h_ref):
    deg = degp_ref[0] + degp_ref[1]
    norm_l = lax.rsqrt(jnp.maximum(deg, 1.0))
    w = wmu_ref[...] + jnp.exp(wlsd_ref[...]) * epsw_ref[...]
    h_ref[...] = jnp.dot(feat_ref[...] * norm_l, w,
                         preferred_element_type=jnp.float32)


_BR1 = 1280

_tc_proj = pl.pallas_call(
    _tc_proj_body,
    grid=(_NPAD // _BR1,),
    in_specs=[
        pl.BlockSpec((_BR1, _D), lambda i: (i, 0)),
        pl.BlockSpec((2, _BR1, 1), lambda i: (0, i, 0)),
        pl.BlockSpec((_D, _D), lambda i: (0, 0)),
        pl.BlockSpec((_D, _D), lambda i: (0, 0)),
        pl.BlockSpec((_D, _D), lambda i: (0, 0)),
    ],
    out_specs=pl.BlockSpec((_BR1, _D), lambda i: (i, 0)),
    out_shape=jax.ShapeDtypeStruct((_NPAD, _D), jnp.float32),
)


def _kl_sum(mu_q, logsd_q, mu_p, logsd_p):
    var_q = jnp.exp(2.0 * logsd_q)
    var_p = jnp.exp(2.0 * logsd_p)
    t = (logsd_p - logsd_q) + (var_q + (mu_q - mu_p) ** 2) / (2.0 * var_p) - 0.5
    return jnp.sum(t)


def _tc_post_body(partials_ref, degp_ref, bmu_ref, blsd_ref, epsb_ref,
                  wmu_ref, wlsd_ref, wpmu_ref, wplsd_ref, bpmu_ref,
                  bplsd_ref, rst_ref, kl_ref):
    i = pl.program_id(0)
    agg = partials_ref[0] + partials_ref[1]
    deg = degp_ref[0] + degp_ref[1]
    norm_r = lax.rsqrt(jnp.maximum(deg, 1.0))
    bias = bmu_ref[...] + jnp.exp(blsd_ref[...]) * epsb_ref[...]
    rst_ref[...] = agg * norm_r + bias

    @pl.when(i == 0)
    def _():
        kl = _kl_sum(wmu_ref[...], wlsd_ref[...], wpmu_ref[...], wplsd_ref[...])
        kl += _kl_sum(bmu_ref[...], blsd_ref[...], bpmu_ref[...], bplsd_ref[...])
        kl_ref[...] = jnp.broadcast_to(kl, (1, 1))


_BR2 = 1000

_tc_post = pl.pallas_call(
    _tc_post_body,
    grid=(_N // _BR2,),
    in_specs=[
        pl.BlockSpec((2, _BR2, _D), lambda i: (0, i, 0)),
        pl.BlockSpec((2, _BR2, 1), lambda i: (0, i, 0)),
        pl.BlockSpec((1, _D), lambda i: (0, 0)),
        pl.BlockSpec((1, _D), lambda i: (0, 0)),
        pl.BlockSpec((1, _D), lambda i: (0, 0)),
        pl.BlockSpec((_D, _D), lambda i: (0, 0)),
        pl.BlockSpec((_D, _D), lambda i: (0, 0)),
        pl.BlockSpec((_D, _D), lambda i: (0, 0)),
        pl.BlockSpec((_D, _D), lambda i: (0, 0)),
        pl.BlockSpec((1, _D), lambda i: (0, 0)),
        pl.BlockSpec((1, _D), lambda i: (0, 0)),
    ],
    out_specs=[
        pl.BlockSpec((_BR2, _D), lambda i: (i, 0)),
        pl.BlockSpec((1, 1), lambda i: (0, 0)),
    ],
    out_shape=[
        jax.ShapeDtypeStruct((_N, _D), jnp.float32),
        jax.ShapeDtypeStruct((1, 1), jnp.float32),
    ],
)


def kernel(feat, edge_index, weight_mu, weight_logsd, bias_mu, bias_logsd,
           weight_prior_mu, weight_prior_logsd, bias_prior_mu,
           bias_prior_logsd, eps_w, eps_b):
    padv = _N + jnp.arange(_EPAD - _E, dtype=jnp.int32) % (_NPAD - _N)
    e = jnp.concatenate([edge_index, jnp.stack([padv, padv])], axis=1)
    edges = e.reshape(2, _NW, _NCHUNK, _CH)
    feat_pad = jnp.concatenate(
        [feat, jnp.zeros((_NPAD - _N, _D), feat.dtype)], axis=0)
    zeros1 = jnp.zeros((_NPAD,), jnp.float32)
    zrow = jnp.zeros((_RPT, _D), jnp.float32)

    degp = _sc_hist(edges, zeros1)
    deg_src = degp[:, 0, :].reshape(_NC, _NPAD, 1)
    deg_dst = degp[:, 1, :].reshape(_NC, _NPAD, 1)

    h = _tc_proj(feat_pad, deg_src, weight_mu, weight_logsd, eps_w)
    partials = _sc_agg(h, edges, zrow)

    rst, kl = _tc_post(partials, deg_dst, bias_mu, bias_logsd, eps_b,
                       weight_mu, weight_logsd, weight_prior_mu,
                       weight_prior_logsd, bias_prior_mu, bias_prior_logsd)
    return rst, kl[0, 0]


# speedup vs baseline: 2.9835x; 1.2140x over previous
import functools

import jax
import jax.numpy as jnp
from jax import lax
from jax.experimental import pallas as pl
from jax.experimental.pallas import tpu as pltpu
from jax.experimental.pallas import tpu_sc as plsc

_N = 10000
_E = 320000
_D = 128
_NC = 2
_NS = 16
_NW = _NC * _NS
_CH = 128
_NCHUNK = 80
_WIN = 8
_NWIN = _NCHUNK // _WIN
_WPW = _NCHUNK * _CH
_EPAD = _NW * _WPW
_NDUMMY = _N
_NPAD = 10240
_RPT = _NPAD // _NS

_mesh = plsc.VectorSubcoreMesh(core_axis_name="c", subcore_axis_name="s")


def _sc_hist_body(edges, zeros1, deg_out, src_idx, dst_idx, ones_v,
                  hist_src, hist_dst):
    c = lax.axis_index("c")
    s = lax.axis_index("s")
    wid = s * _NC + c
    for i in range(_CH // 16):
        ones_v[pl.ds(i * 16, 16)] = jnp.ones((16,), jnp.float32)
    @pl.when(s == 0)
    def _():
        pltpu.sync_copy(zeros1, hist_src)
        pltpu.sync_copy(zeros1, hist_dst)
    pltpu.sync_copy(edges.at[0, wid], src_idx)
    pltpu.sync_copy(edges.at[1, wid], dst_idx)
    plsc.subcore_barrier()

    @pl.loop(0, _NCHUNK)
    def _(j):
        pltpu.sync_copy(ones_v, hist_src.at[src_idx.at[j]], add=True)
        pltpu.sync_copy(ones_v, hist_dst.at[dst_idx.at[j]], add=True)

    plsc.subcore_barrier()
    @pl.when(s == 0)
    def _():
        pltpu.sync_copy(hist_src, deg_out.at[c, 0])
        pltpu.sync_copy(hist_dst, deg_out.at[c, 1])


_sc_hist = pl.kernel(
    _sc_hist_body,
    out_type=jax.ShapeDtypeStruct((_NC, 2, _NPAD), jnp.float32),
    mesh=_mesh,
    scratch_types=[
        pltpu.VMEM((_NCHUNK, _CH), jnp.int32),
        pltpu.VMEM((_NCHUNK, _CH), jnp.int32),
        pltpu.VMEM((_CH,), jnp.float32),
        pltpu.VMEM_SHARED((_NPAD,), jnp.float32),
        pltpu.VMEM_SHARED((_NPAD,), jnp.float32),
    ],
)


_HALF = _NCHUNK // 2


def _sc_agg_body(h, edges, zrow, out, src_idx, dst_idx, rows, agg_sh,
                 gsem, ssem):
    c = lax.axis_index("c")
    s = lax.axis_index("s")
    wid = s * _NC + c
    pltpu.sync_copy(zrow, agg_sh.at[pl.ds(s * _RPT, _RPT)])
    plsc.subcore_barrier()

    def _phase(base):
        pltpu.sync_copy(edges.at[0, wid, pl.ds(base, _HALF)], src_idx)
        pltpu.sync_copy(edges.at[1, wid, pl.ds(base, _HALF)], dst_idx)
        pltpu.async_copy(h.at[src_idx.at[0]], rows.at[0], gsem).wait()

        @pl.loop(0, _HALF - 1)
        def _(j):
            b = j % 2
            dg = pltpu.async_copy(h.at[src_idx.at[j + 1]], rows.at[1 - b],
                                  gsem)
            ds = pltpu.async_copy(rows.at[b], agg_sh.at[dst_idx.at[j]],
                                  ssem, add=True)
            ds.wait()
            dg.wait()

        bl = (_HALF - 1) % 2
        pltpu.async_copy(rows.at[bl], agg_sh.at[dst_idx.at[_HALF - 1]],
                         ssem, add=True).wait()

    _phase(0)
    _phase(_HALF)

    plsc.subcore_barrier()
    pltpu.sync_copy(agg_sh.at[pl.ds(s * _RPT, _RPT)],
                    out.at[c, pl.ds(s * _RPT, _RPT)])


_sc_agg = pl.kernel(
    _sc_agg_body,
    out_type=jax.ShapeDtypeStruct((_NC, _NPAD, _D), jnp.float32),
    mesh=_mesh,
    scratch_types=[
        pltpu.VMEM((_HALF, _CH), jnp.int32),
        pltpu.VMEM((_HALF, _CH), jnp.int32),
        pltpu.VMEM((2, _CH, _D), jnp.float32),
        pltpu.VMEM_SHARED((_NPAD, _D), jnp.float32),
        pltpu.SemaphoreType.DMA,
        pltpu.SemaphoreType.DMA,
    ],
)


def _tc_proj_body(feat_ref, degp_ref, wmu_ref, wlsd_ref, epsw_ref, h_ref):
    deg = degp_ref[0] + degp_ref[1]
    norm_l = lax.rsqrt(jnp.maximum(deg, 1.0))
    w = wmu_ref[...] + jnp.exp(wlsd_ref[...]) * epsw_ref[...]
    h_ref[...] = jnp.dot(feat_ref[...] * norm_l, w,
                         preferred_element_type=jnp.float32)


_BR1 = 1280

_tc_proj = pl.pallas_call(
    _tc_proj_body,
    grid=(_NPAD // _BR1,),
    in_specs=[
        pl.BlockSpec((_BR1, _D), lambda i: (i, 0)),
        pl.BlockSpec((2, _BR1, 1), lambda i: (0, i, 0)),
        pl.BlockSpec((_D, _D), lambda i: (0, 0)),
        pl.BlockSpec((_D, _D), lambda i: (0, 0)),
        pl.BlockSpec((_D, _D), lambda i: (0, 0)),
    ],
    out_specs=pl.BlockSpec((_BR1, _D), lambda i: (i, 0)),
    out_shape=jax.ShapeDtypeStruct((_NPAD, _D), jnp.float32),
)


def _kl_sum(mu_q, logsd_q, mu_p, logsd_p):
    var_q = jnp.exp(2.0 * logsd_q)
    var_p = jnp.exp(2.0 * logsd_p)
    t = (logsd_p - logsd_q) + (var_q + (mu_q - mu_p) ** 2) / (2.0 * var_p) - 0.5
    return jnp.sum(t)


def _tc_post_body(partials_ref, degp_ref, bmu_ref, blsd_ref, epsb_ref,
                  wmu_ref, wlsd_ref, wpmu_ref, wplsd_ref, bpmu_ref,
                  bplsd_ref, rst_ref, kl_ref):
    i = pl.program_id(0)
    agg = partials_ref[0] + partials_ref[1]
    deg = degp_ref[0] + degp_ref[1]
    norm_r = lax.rsqrt(jnp.maximum(deg, 1.0))
    bias = bmu_ref[...] + jnp.exp(blsd_ref[...]) * epsb_ref[...]
    rst_ref[...] = agg * norm_r + bias

    @pl.when(i == 0)
    def _():
        kl = _kl_sum(wmu_ref[...], wlsd_ref[...], wpmu_ref[...], wplsd_ref[...])
        kl += _kl_sum(bmu_ref[...], blsd_ref[...], bpmu_ref[...], bplsd_ref[...])
        kl_ref[...] = jnp.broadcast_to(kl, (1, 1))


_BR2 = 1000

_tc_post = pl.pallas_call(
    _tc_post_body,
    grid=(_N // _BR2,),
    in_specs=[
        pl.BlockSpec((2, _BR2, _D), lambda i: (0, i, 0)),
        pl.BlockSpec((2, _BR2, 1), lambda i: (0, i, 0)),
        pl.BlockSpec((1, _D), lambda i: (0, 0)),
        pl.BlockSpec((1, _D), lambda i: (0, 0)),
        pl.BlockSpec((1, _D), lambda i: (0, 0)),
        pl.BlockSpec((_D, _D), lambda i: (0, 0)),
        pl.BlockSpec((_D, _D), lambda i: (0, 0)),
        pl.BlockSpec((_D, _D), lambda i: (0, 0)),
        pl.BlockSpec((_D, _D), lambda i: (0, 0)),
        pl.BlockSpec((1, _D), lambda i: (0, 0)),
        pl.BlockSpec((1, _D), lambda i: (0, 0)),
    ],
    out_specs=[
        pl.BlockSpec((_BR2, _D), lambda i: (i, 0)),
        pl.BlockSpec((1, 1), lambda i: (0, 0)),
    ],
    out_shape=[
        jax.ShapeDtypeStruct((_N, _D), jnp.float32),
        jax.ShapeDtypeStruct((1, 1), jnp.float32),
    ],
)


def kernel(feat, edge_index, weight_mu, weight_logsd, bias_mu, bias_logsd,
           weight_prior_mu, weight_prior_logsd, bias_prior_mu,
           bias_prior_logsd, eps_w, eps_b):
    padv = _N + jnp.arange(_EPAD - _E, dtype=jnp.int32) % (_NPAD - _N)
    e = jnp.concatenate([edge_index, jnp.stack([padv, padv])], axis=1)
    edges = e.reshape(2, _NW, _NCHUNK, _CH)
    feat_pad = jnp.concatenate(
        [feat, jnp.zeros((_NPAD - _N, _D), feat.dtype)], axis=0)
    zeros1 = jnp.zeros((_NPAD,), jnp.float32)
    zrow = jnp.zeros((_RPT, _D), jnp.float32)

    degp = _sc_hist(edges, zeros1)
    deg_src = degp[:, 0, :].reshape(_NC, _NPAD, 1)
    deg_dst = degp[:, 1, :].reshape(_NC, _NPAD, 1)

    h = _tc_proj(feat_pad, deg_src, weight_mu, weight_logsd, eps_w)
    partials = _sc_agg(h, edges, zrow)

    rst, kl = _tc_post(partials, deg_dst, bias_mu, bias_logsd, eps_b,
                       weight_mu, weight_logsd, weight_prior_mu,
                       weight_prior_logsd, bias_prior_mu, bias_prior_logsd)
    return rst, kl[0, 0]


# speedup vs baseline: 3.0293x; 1.0153x over previous
import functools

import jax
import jax.numpy as jnp
from jax import lax
from jax.experimental import pallas as pl
from jax.experimental.pallas import tpu as pltpu
from jax.experimental.pallas import tpu_sc as plsc

_N = 10000
_E = 320000
_D = 128
_NC = 2
_NS = 16
_NW = _NC * _NS
_CH = 128
_NCHUNK = 80
_WIN = 8
_NWIN = _NCHUNK // _WIN
_WPW = _NCHUNK * _CH
_EPAD = _NW * _WPW
_NDUMMY = _N
_NPAD = 10240
_RPT = _NPAD // _NS

_mesh = plsc.VectorSubcoreMesh(core_axis_name="c", subcore_axis_name="s")


def _sc_hist_body(edges, zeros1, deg_out, src_idx, dst_idx, ones_v,
                  hist_src, hist_dst, sem_a, sem_b):
    c = lax.axis_index("c")
    s = lax.axis_index("s")
    wid = s * _NC + c
    for i in range(_CH // 16):
        ones_v[pl.ds(i * 16, 16)] = jnp.ones((16,), jnp.float32)
    @pl.when(s == 0)
    def _():
        pltpu.sync_copy(zeros1, hist_src)
        pltpu.sync_copy(zeros1, hist_dst)
    pltpu.sync_copy(edges.at[0, wid], src_idx)
    pltpu.sync_copy(edges.at[1, wid], dst_idx)
    plsc.subcore_barrier()

    @pl.loop(0, _NCHUNK)
    def _(j):
        da = pltpu.async_copy(ones_v, hist_src.at[src_idx.at[j]], sem_a,
                              add=True)
        db = pltpu.async_copy(ones_v, hist_dst.at[dst_idx.at[j]], sem_b,
                              add=True)
        da.wait()
        db.wait()

    plsc.subcore_barrier()
    @pl.when(s == 0)
    def _():
        pltpu.sync_copy(hist_src, deg_out.at[c, 0])
        pltpu.sync_copy(hist_dst, deg_out.at[c, 1])


_sc_hist = pl.kernel(
    _sc_hist_body,
    out_type=jax.ShapeDtypeStruct((_NC, 2, _NPAD), jnp.float32),
    mesh=_mesh,
    scratch_types=[
        pltpu.VMEM((_NCHUNK, _CH), jnp.int32),
        pltpu.VMEM((_NCHUNK, _CH), jnp.int32),
        pltpu.VMEM((_CH,), jnp.float32),
        pltpu.VMEM_SHARED((_NPAD,), jnp.float32),
        pltpu.VMEM_SHARED((_NPAD,), jnp.float32),
        pltpu.SemaphoreType.DMA,
        pltpu.SemaphoreType.DMA,
    ],
)


_HALF = _NCHUNK // 2


def _sc_agg_body(h, edges, zrow, out, src_idx, dst_idx, rows, agg_sh,
                 gsem, ssem):
    c = lax.axis_index("c")
    s = lax.axis_index("s")
    wid = s * _NC + c
    pltpu.sync_copy(zrow, agg_sh.at[pl.ds(s * _RPT, _RPT)])
    plsc.subcore_barrier()

    def _phase(base):
        pltpu.sync_copy(edges.at[0, wid, pl.ds(base, _HALF)], src_idx)
        pltpu.sync_copy(edges.at[1, wid, pl.ds(base, _HALF)], dst_idx)
        pltpu.async_copy(h.at[src_idx.at[0]], rows.at[0], gsem).wait()

        @pl.loop(0, _HALF - 1)
        def _(j):
            b = j % 2
            dg = pltpu.async_copy(h.at[src_idx.at[j + 1]], rows.at[1 - b],
                                  gsem)
            ds = pltpu.async_copy(rows.at[b], agg_sh.at[dst_idx.at[j]],
                                  ssem, add=True)
            ds.wait()
            dg.wait()

        bl = (_HALF - 1) % 2
        pltpu.async_copy(rows.at[bl], agg_sh.at[dst_idx.at[_HALF - 1]],
                         ssem, add=True).wait()

    _phase(0)
    _phase(_HALF)

    plsc.subcore_barrier()
    pltpu.sync_copy(agg_sh.at[pl.ds(s * _RPT, _RPT)],
                    out.at[c, pl.ds(s * _RPT, _RPT)])


_sc_agg = pl.kernel(
    _sc_agg_body,
    out_type=jax.ShapeDtypeStruct((_NC, _NPAD, _D), jnp.float32),
    mesh=_mesh,
    scratch_types=[
        pltpu.VMEM((_HALF, _CH), jnp.int32),
        pltpu.VMEM((_HALF, _CH), jnp.int32),
        pltpu.VMEM((2, _CH, _D), jnp.float32),
        pltpu.VMEM_SHARED((_NPAD, _D), jnp.float32),
        pltpu.SemaphoreType.DMA,
        pltpu.SemaphoreType.DMA,
    ],
)


def _tc_proj_body(feat_ref, degp_ref, wmu_ref, wlsd_ref, epsw_ref, h_ref):
    deg = degp_ref[0] + degp_ref[1]
    norm_l = lax.rsqrt(jnp.maximum(deg, 1.0))
    w = wmu_ref[...] + jnp.exp(wlsd_ref[...]) * epsw_ref[...]
    h_ref[...] = jnp.dot(feat_ref[...] * norm_l, w,
                         preferred_element_type=jnp.float32)


_BR1 = 1280

_tc_proj = pl.pallas_call(
    _tc_proj_body,
    grid=(_NPAD // _BR1,),
    in_specs=[
        pl.BlockSpec((_BR1, _D), lambda i: (i, 0)),
        pl.BlockSpec((2, _BR1, 1), lambda i: (0, i, 0)),
        pl.BlockSpec((_D, _D), lambda i: (0, 0)),
        pl.BlockSpec((_D, _D), lambda i: (0, 0)),
        pl.BlockSpec((_D, _D), lambda i: (0, 0)),
    ],
    out_specs=pl.BlockSpec((_BR1, _D), lambda i: (i, 0)),
    out_shape=jax.ShapeDtypeStruct((_NPAD, _D), jnp.float32),
)


def _kl_sum(mu_q, logsd_q, mu_p, logsd_p):
    var_q = jnp.exp(2.0 * logsd_q)
    var_p = jnp.exp(2.0 * logsd_p)
    t = (logsd_p - logsd_q) + (var_q + (mu_q - mu_p) ** 2) / (2.0 * var_p) - 0.5
    return jnp.sum(t)


def _tc_post_body(partials_ref, degp_ref, bmu_ref, blsd_ref, epsb_ref,
                  wmu_ref, wlsd_ref, wpmu_ref, wplsd_ref, bpmu_ref,
                  bplsd_ref, rst_ref, kl_ref):
    i = pl.program_id(0)
    agg = partials_ref[0] + partials_ref[1]
    deg = degp_ref[0] + degp_ref[1]
    norm_r = lax.rsqrt(jnp.maximum(deg, 1.0))
    bias = bmu_ref[...] + jnp.exp(blsd_ref[...]) * epsb_ref[...]
    rst_ref[...] = agg * norm_r + bias

    @pl.when(i == 0)
    def _():
        kl = _kl_sum(wmu_ref[...], wlsd_ref[...], wpmu_ref[...], wplsd_ref[...])
        kl += _kl_sum(bmu_ref[...], blsd_ref[...], bpmu_ref[...], bplsd_ref[...])
        kl_ref[...] = jnp.broadcast_to(kl, (1, 1))


_BR2 = 1000

_tc_post = pl.pallas_call(
    _tc_post_body,
    grid=(_N // _BR2,),
    in_specs=[
        pl.BlockSpec((2, _BR2, _D), lambda i: (0, i, 0)),
        pl.BlockSpec((2, _BR2, 1), lambda i: (0, i, 0)),
        pl.BlockSpec((1, _D), lambda i: (0, 0)),
        pl.BlockSpec((1, _D), lambda i: (0, 0)),
        pl.BlockSpec((1, _D), lambda i: (0, 0)),
        pl.BlockSpec((_D, _D), lambda i: (0, 0)),
        pl.BlockSpec((_D, _D), lambda i: (0, 0)),
        pl.BlockSpec((_D, _D), lambda i: (0, 0)),
        pl.BlockSpec((_D, _D), lambda i: (0, 0)),
        pl.BlockSpec((1, _D), lambda i: (0, 0)),
        pl.BlockSpec((1, _D), lambda i: (0, 0)),
    ],
    out_specs=[
        pl.BlockSpec((_BR2, _D), lambda i: (i, 0)),
        pl.BlockSpec((1, 1), lambda i: (0, 0)),
    ],
    out_shape=[
        jax.ShapeDtypeStruct((_N, _D), jnp.float32),
        jax.ShapeDtypeStruct((1, 1), jnp.float32),
    ],
)


def kernel(feat, edge_index, weight_mu, weight_logsd, bias_mu, bias_logsd,
           weight_prior_mu, weight_prior_logsd, bias_prior_mu,
           bias_prior_logsd, eps_w, eps_b):
    padv = _N + jnp.arange(_EPAD - _E, dtype=jnp.int32) % (_NPAD - _N)
    e = jnp.concatenate([edge_index, jnp.stack([padv, padv])], axis=1)
    edges = e.reshape(2, _NW, _NCHUNK, _CH)
    feat_pad = jnp.concatenate(
        [feat, jnp.zeros((_NPAD - _N, _D), feat.dtype)], axis=0)
    zeros1 = jnp.zeros((_NPAD,), jnp.float32)
    zrow = jnp.zeros((_RPT, _D), jnp.float32)

    degp = _sc_hist(edges, zeros1)
    deg_src = degp[:, 0, :].reshape(_NC, _NPAD, 1)
    deg_dst = degp[:, 1, :].reshape(_NC, _NPAD, 1)

    h = _tc_proj(feat_pad, deg_src, weight_mu, weight_logsd, eps_w)
    partials = _sc_agg(h, edges, zrow)

    rst, kl = _tc_post(partials, deg_dst, bias_mu, bias_logsd, eps_b,
                       weight_mu, weight_logsd, weight_prior_mu,
                       weight_prior_logsd, bias_prior_mu, bias_prior_logsd)
    return rst, kl[0, 0]
